# baseline (speedup 1.0000x reference)
import functools

import jax
import jax.numpy as jnp
from jax import lax
from jax.experimental import pallas as pl
from jax.experimental.pallas import tpu as pltpu
from jax.experimental.pallas import tpu_sc as plsc

VOCAB = 1000000
CHAR_DIM = 32
LATENT = 32
HIDDEN = 128
BATCH = 16384

_NC = 2
_NS = 16
_NW = _NC * _NS
_B_PER_W = BATCH // _NW
_CHUNK = 128
_NCHUNK = _B_PER_W // _CHUNK

_TC_BLK = 2048


def _sc_gather(table, idx3):
    mesh = plsc.VectorSubcoreMesh(
        core_axis_name="c", subcore_axis_name="s",
        num_cores=_NC, num_subcores=_NS,
    )

    @functools.partial(
        pl.kernel,
        out_type=jax.ShapeDtypeStruct((BATCH, CHAR_DIM), jnp.float32),
        mesh=mesh,
        scratch_types=[
            pltpu.VMEM((_NCHUNK, _CHUNK), jnp.int32),
            pltpu.VMEM((_B_PER_W, CHAR_DIM), jnp.float32),
            pltpu.SemaphoreType.DMA,
        ],
        compiler_params=pltpu.CompilerParams(use_tc_tiling_on_sc=False),
    )
    def gather_kernel(table_hbm, idx_hbm, out_hbm, idx_v, rows_v, sem):
        wid = lax.axis_index("s") * _NC + lax.axis_index("c")
        pltpu.sync_copy(idx_hbm.at[wid], idx_v)
        copies = [
            pltpu.async_copy(
                table_hbm.at[idx_v.at[j]],
                rows_v.at[pl.ds(j * _CHUNK, _CHUNK)],
                sem,
            )
            for j in range(_NCHUNK)
        ]
        for cp in copies:
            cp.wait()
        pltpu.sync_copy(rows_v, out_hbm.at[pl.ds(wid * _B_PER_W, _B_PER_W)])

    return gather_kernel(table, idx3)


def _vae_body(x_ref, eps_ref, encW, encb, muW, mub, varW, varb,
              dinW, dinb, decW, decb, finW, finb,
              rec_ref, mu_ref, lv_ref):
    def leaky(a):
        return jnp.where(a > 0, a, 0.01 * a)

    x = x_ref[...]
    h = jnp.dot(x, encW[...], preferred_element_type=jnp.float32) + encb[...]
    h = leaky(h)
    mu = jnp.dot(h, muW[...], preferred_element_type=jnp.float32) + mub[...]
    lv = jnp.dot(h, varW[...], preferred_element_type=jnp.float32) + varb[...]
    z = eps_ref[...] * jnp.exp(0.5 * lv) + mu
    d = jnp.dot(z, dinW[...], preferred_element_type=jnp.float32) + dinb[...]
    d = leaky(d)
    d = jnp.dot(d, decW[...], preferred_element_type=jnp.float32) + decb[...]
    d = leaky(d)
    rec_ref[...] = jnp.dot(d, finW[...], preferred_element_type=jnp.float32) + finb[...]
    mu_ref[...] = mu
    lv_ref[...] = lv


def _vae_chain(x, eps, enc_W, enc_b, mu_W, mu_b, var_W, var_b,
               din_W, din_b, dec_W, dec_b, fin_W, fin_b):
    grid = (BATCH // _TC_BLK,)
    blk = lambda r, c: pl.BlockSpec((_TC_BLK, c), lambda i: (i, 0))
    full = lambda a: pl.BlockSpec(a.shape, lambda i: (0,) * a.ndim)
    weights = (enc_W, enc_b.reshape(1, HIDDEN), mu_W, mu_b.reshape(1, LATENT),
               var_W, var_b.reshape(1, LATENT), din_W, din_b.reshape(1, HIDDEN),
               dec_W, dec_b.reshape(1, LATENT), fin_W, fin_b.reshape(1, CHAR_DIM))
    return pl.pallas_call(
        _vae_body,
        grid=grid,
        in_specs=[blk(_TC_BLK, CHAR_DIM), blk(_TC_BLK, LATENT)]
                 + [full(w) for w in weights],
        out_specs=[blk(_TC_BLK, CHAR_DIM), blk(_TC_BLK, LATENT),
                   blk(_TC_BLK, LATENT)],
        out_shape=[
            jax.ShapeDtypeStruct((BATCH, CHAR_DIM), jnp.float32),
            jax.ShapeDtypeStruct((BATCH, LATENT), jnp.float32),
            jax.ShapeDtypeStruct((BATCH, LATENT), jnp.float32),
        ],
    )(x, eps, *weights)


def kernel(indices, table, enc_W, enc_b, mu_W, mu_b, var_W, var_b,
           din_W, din_b, dec_W, dec_b, fin_W, fin_b):
    idx3 = indices.reshape(_NW, _NCHUNK, _CHUNK)
    x = _sc_gather(table, idx3)
    eps = jax.random.normal(jax.random.key(42), (BATCH, LATENT), dtype=jnp.float32)
    recons, mu, log_var = _vae_chain(
        x, eps, enc_W, enc_b, mu_W, mu_b, var_W, var_b,
        din_W, din_b, dec_W, dec_b, fin_W, fin_b)
    return (recons, x, mu, log_var)


# speedup vs baseline: 1.5387x; 1.5387x over previous
import functools

import jax
import jax.numpy as jnp
from jax import lax
from jax.experimental import pallas as pl
from jax.experimental.pallas import tpu as pltpu
from jax.experimental.pallas import tpu_sc as plsc

VOCAB = 1000000
CHAR_DIM = 32
LATENT = 32
HIDDEN = 128
BATCH = 16384

_NC = 2
_NS = 16
_NW = _NC * _NS
_B_PER_W = BATCH // _NW
_CHUNK = 128
_NCHUNK = _B_PER_W // _CHUNK

_TC_BLK = 2048


def _sc_gather(table, indices):
    mesh = plsc.VectorSubcoreMesh(
        core_axis_name="c", subcore_axis_name="s",
        num_cores=_NC, num_subcores=_NS,
    )

    @functools.partial(
        pl.kernel,
        out_type=jax.ShapeDtypeStruct((BATCH, CHAR_DIM), jnp.float32),
        mesh=mesh,
        scratch_types=[
            pltpu.VMEM((_B_PER_W,), jnp.int32),
            pltpu.VMEM((_B_PER_W, CHAR_DIM), jnp.float32),
            pltpu.SemaphoreType.DMA,
        ],
    )
    def gather_kernel(table_hbm, idx_hbm, out_hbm, idx_v, rows_v, sem):
        wid = lax.axis_index("s") * _NC + lax.axis_index("c")
        base = wid * _B_PER_W
        pltpu.sync_copy(idx_hbm.at[pl.ds(base, _B_PER_W)], idx_v)

        def fire_group(g, carry):
            vec = idx_v[pl.ds(g * 16, 16)]
            for k in range(16):
                pltpu.async_copy(
                    table_hbm.at[vec[k]], rows_v.at[g * 16 + k], sem)
            return carry

        lax.fori_loop(0, _B_PER_W // 16, fire_group, 0)
        pltpu.make_async_copy(table_hbm.at[pl.ds(0, _B_PER_W)], rows_v, sem).wait()
        pltpu.sync_copy(rows_v, out_hbm.at[pl.ds(base, _B_PER_W)])

    return gather_kernel(table, indices)


def _vae_body(x_ref, eps_ref, encW, encb, muW, mub, varW, varb,
              dinW, dinb, decW, decb, finW, finb,
              rec_ref, mu_ref, lv_ref):
    def leaky(a):
        return jnp.where(a > 0, a, 0.01 * a)

    x = x_ref[...]
    h = jnp.dot(x, encW[...], preferred_element_type=jnp.float32) + encb[...]
    h = leaky(h)
    mu = jnp.dot(h, muW[...], preferred_element_type=jnp.float32) + mub[...]
    lv = jnp.dot(h, varW[...], preferred_element_type=jnp.float32) + varb[...]
    z = eps_ref[...] * jnp.exp(0.5 * lv) + mu
    d = jnp.dot(z, dinW[...], preferred_element_type=jnp.float32) + dinb[...]
    d = leaky(d)
    d = jnp.dot(d, decW[...], preferred_element_type=jnp.float32) + decb[...]
    d = leaky(d)
    rec_ref[...] = jnp.dot(d, finW[...], preferred_element_type=jnp.float32) + finb[...]
    mu_ref[...] = mu
    lv_ref[...] = lv


def _vae_chain(x, eps, enc_W, enc_b, mu_W, mu_b, var_W, var_b,
               din_W, din_b, dec_W, dec_b, fin_W, fin_b):
    grid = (BATCH // _TC_BLK,)
    blk = lambda r, c: pl.BlockSpec((_TC_BLK, c), lambda i: (i, 0))
    full = lambda a: pl.BlockSpec(a.shape, lambda i: (0,) * a.ndim)
    weights = (enc_W, enc_b.reshape(1, HIDDEN), mu_W, mu_b.reshape(1, LATENT),
               var_W, var_b.reshape(1, LATENT), din_W, din_b.reshape(1, HIDDEN),
               dec_W, dec_b.reshape(1, LATENT), fin_W, fin_b.reshape(1, CHAR_DIM))
    return pl.pallas_call(
        _vae_body,
        grid=grid,
        in_specs=[blk(_TC_BLK, CHAR_DIM), blk(_TC_BLK, LATENT)]
                 + [full(w) for w in weights],
        out_specs=[blk(_TC_BLK, CHAR_DIM), blk(_TC_BLK, LATENT),
                   blk(_TC_BLK, LATENT)],
        out_shape=[
            jax.ShapeDtypeStruct((BATCH, CHAR_DIM), jnp.float32),
            jax.ShapeDtypeStruct((BATCH, LATENT), jnp.float32),
            jax.ShapeDtypeStruct((BATCH, LATENT), jnp.float32),
        ],
    )(x, eps, *weights)


def kernel(indices, table, enc_W, enc_b, mu_W, mu_b, var_W, var_b,
           din_W, din_b, dec_W, dec_b, fin_W, fin_b):
    x = _sc_gather(table, indices)
    eps = jax.random.normal(jax.random.key(42), (BATCH, LATENT), dtype=jnp.float32)
    recons, mu, log_var = _vae_chain(
        x, eps, enc_W, enc_b, mu_W, mu_b, var_W, var_b,
        din_W, din_b, dec_W, dec_b, fin_W, fin_b)
    return (recons, x, mu, log_var)
